# baseline (speedup 1.0000x reference)
import functools

import jax
import jax.numpy as jnp
from jax import lax
from jax.experimental import pallas as pl
from jax.experimental.pallas import tpu as pltpu
from jax.experimental.pallas import tpu_sc as plsc

B = 4096
NNEG = 256
D = 64
NCHUNK = 2
CHUNK = NNEG // NCHUNK
L = 16
NW = 32
BPW = B // NW


def _dist_mult_sc(hid, rid, tid, negh, negt, ent, rel,
                  fact_o, hs_o, ts_o,
                  hid_v, rid_v, tid_v,
                  rt_v, hr_v, factp_v, fact_v,
                  nidxA, nidxB, rowsA0, rowsB0, rowsA1, rowsB1, score_v,
                  semA0, semB0, semA1, semB1):
    wid = lax.axis_index("c") * 16 + lax.axis_index("s")
    base = wid * BPW

    f32 = jnp.float32
    i32 = jnp.int32
    iota = lax.iota(i32, L)

    head_v, relr_v, tail_v = rowsA0, rowsB0, rowsA1
    pltpu.sync_copy(hid.at[pl.ds(base, BPW)], hid_v)
    pltpu.sync_copy(rid.at[pl.ds(base, BPW)], rid_v)
    pltpu.sync_copy(tid.at[pl.ds(base, BPW)], tid_v)
    pltpu.async_copy(ent.at[hid_v], head_v, semA0).wait()
    pltpu.async_copy(rel.at[rid_v], relr_v, semA0).wait()
    pltpu.async_copy(ent.at[tid_v], tail_v, semA0).wait()

    UNP = plsc.PackFormat.INTERLEAVED

    def pbody(b, carry):
        facc = jnp.zeros((L,), f32)
        for k in range(2):
            sl = pl.ds(k * 2 * L, 2 * L)
            he, ho = plsc.unpack(head_v[b, sl], format=UNP)
            re, ro = plsc.unpack(relr_v[b, sl], format=UNP)
            te, to = plsc.unpack(tail_v[b, sl], format=UNP)
            rte = re * te
            rto = ro * to
            rt_v[b, pl.ds(2 * k * L, L)] = rte
            rt_v[b, pl.ds((2 * k + 1) * L, L)] = rto
            hr_v[b, pl.ds(2 * k * L, L)] = he * re
            hr_v[b, pl.ds((2 * k + 1) * L, L)] = ho * ro
            facc = facc + he * rte + ho * rto
        factp_v[pl.ds(b * L, L)] = facc
        return carry

    lax.fori_loop(0, BPW, pbody, 0)
    gbase = [(iota + g * L) * L for g in range(BPW // L)]
    rot = [jnp.bitwise_and(iota + d, L - 1) for d in range(L)]
    for q in range(BPW // L):
        acc = jnp.zeros((L,), f32)
        for d in range(L):
            acc = acc + plsc.load_gather(factp_v, [gbase[q] + rot[d]])
        fact_v[pl.ds(q * L, L)] = acc
    pltpu.sync_copy(fact_v, fact_o.at[pl.ds(base, BPW)])

    bufs = ((rowsA0, semA0, rowsB0, semB0), (rowsA1, semA1, rowsB1, semB1))

    def fire(b, s):
        rA, sA, rB, sB = bufs[s]
        pltpu.async_copy(ent.at[nidxA.at[b]], rA, sA)
        pltpu.async_copy(ent.at[nidxB.at[b]], rB, sB)

    def drain(s):
        rA, sA, rB, sB = bufs[s]
        pltpu.make_async_copy(ent.at[nidxA.at[0]], rA, sA).wait()
        pltpu.make_async_copy(ent.at[nidxB.at[0]], rB, sB).wait()

    def side(neg, coef_v, out_hbm):
        pltpu.sync_copy(neg.at[pl.ds(base, BPW), pl.ds(0, CHUNK)], nidxA)
        pltpu.sync_copy(neg.at[pl.ds(base, BPW), pl.ds(CHUNK, CHUNK)], nidxB)
        fire(0, 0)

        def compute(b, s):
            rA, _, rB, _ = bufs[s]
            cvecs = [coef_v[b, pl.ds(k * L, L)] for k in range(D // L)]
            for j, rows in ((0, rA), (1, rB)):
                def rbody(n, carry, rows=rows):
                    e0, o0 = plsc.unpack(rows[n, pl.ds(0, 2 * L)], format=UNP)
                    e1, o1 = plsc.unpack(rows[n, pl.ds(2 * L, 2 * L)], format=UNP)
                    facc = (e0 * cvecs[0] + o0 * cvecs[1]
                            + e1 * cvecs[2] + o1 * cvecs[3])
                    factp_v[pl.ds(n * L, L)] = facc
                    return carry

                lax.fori_loop(0, CHUNK, rbody, 0, unroll=4)
                accs = [jnp.zeros((L,), f32) for _ in range(CHUNK // L)]
                for d in range(L):
                    r = rot[d]
                    for g in range(CHUNK // L):
                        accs[g] = accs[g] + plsc.load_gather(
                            factp_v, [gbase[g] + r])
                for g in range(CHUNK // L):
                    score_v[b, pl.ds(j * CHUNK + g * L, L)] = accs[g]

        def pair(i, carry):
            b0 = 2 * i
            b1 = b0 + 1
            fire(b1, 1)
            drain(0)
            compute(b0, 0)
            fire(jnp.minimum(b1 + 1, BPW - 1), 0)
            drain(1)
            compute(b1, 1)
            return carry

        lax.fori_loop(0, BPW // 2, pair, 0)
        drain(0)
        pltpu.sync_copy(score_v, out_hbm.at[pl.ds(base, BPW), :])

    side(negh, rt_v, hs_o)
    side(negt, hr_v, ts_o)


def kernel(positive_samples, negative_heads, negative_tails,
           entity_embeddings, relation_embeddings):
    i32 = jnp.int32
    hid = positive_samples[:, 0].astype(i32)
    rid = positive_samples[:, 1].astype(i32)
    tid = positive_samples[:, 2].astype(i32)
    negh = negative_heads.astype(i32)
    negt = negative_tails.astype(i32)
    entb = entity_embeddings.astype(jnp.bfloat16)
    relb = relation_embeddings.astype(jnp.bfloat16)

    mesh = plsc.VectorSubcoreMesh(core_axis_name="c", subcore_axis_name="s")
    run = functools.partial(
        pl.kernel,
        mesh=mesh,
        compiler_params=pltpu.CompilerParams(
            needs_layout_passes=False, use_tc_tiling_on_sc=False),
        out_type=[
            jax.ShapeDtypeStruct((B,), jnp.float32),
            jax.ShapeDtypeStruct((B, NNEG), jnp.float32),
            jax.ShapeDtypeStruct((B, NNEG), jnp.float32),
        ],
        scratch_types=[
            pltpu.VMEM((BPW,), i32),
            pltpu.VMEM((BPW,), i32),
            pltpu.VMEM((BPW,), i32),
            pltpu.VMEM((BPW, D), jnp.float32),
            pltpu.VMEM((BPW, D), jnp.float32),
            pltpu.VMEM((BPW * L,), jnp.float32),
            pltpu.VMEM((BPW,), jnp.float32),
            pltpu.VMEM((BPW, CHUNK), i32),
            pltpu.VMEM((BPW, CHUNK), i32),
            pltpu.VMEM((CHUNK, D), jnp.bfloat16),
            pltpu.VMEM((CHUNK, D), jnp.bfloat16),
            pltpu.VMEM((CHUNK, D), jnp.bfloat16),
            pltpu.VMEM((CHUNK, D), jnp.bfloat16),
            pltpu.VMEM((BPW, NNEG), jnp.float32),
            pltpu.SemaphoreType.DMA,
            pltpu.SemaphoreType.DMA,
            pltpu.SemaphoreType.DMA,
            pltpu.SemaphoreType.DMA,
        ],
    )(_dist_mult_sc)

    fact, hs, ts = run(hid, rid, tid, negh, negt, entb, relb)
    return (fact.reshape(B, 1), hs, ts)


# speedup vs baseline: 1.1464x; 1.1464x over previous
import functools

import jax
import jax.numpy as jnp
from jax import lax
from jax.experimental import pallas as pl
from jax.experimental.pallas import tpu as pltpu
from jax.experimental.pallas import tpu_sc as plsc

B = 4096
NNEG = 256
D = 64
NCHUNK = 2
CHUNK = NNEG // NCHUNK
L = 16
NW = 32
BPW = B // NW


def _dist_mult_sc(hid, rid, tid, negh, negt, ent, rel,
                  fact_o, hs_o, ts_o,
                  hid_v, rid_v, tid_v,
                  rt_v, hr_v, factp_v, fact_v,
                  nidxA, nidxB, rowsA0, rowsB0, rowsA1, rowsB1, score_v,
                  semA0, semB0, semA1, semB1):
    wid = lax.axis_index("c") * 16 + lax.axis_index("s")
    base = wid * BPW

    f32 = jnp.float32
    i32 = jnp.int32
    iota = lax.iota(i32, L)

    head_v, relr_v, tail_v = rowsA0, rowsB0, rowsA1
    pltpu.sync_copy(hid.at[pl.ds(base, BPW)], hid_v)
    pltpu.sync_copy(rid.at[pl.ds(base, BPW)], rid_v)
    pltpu.sync_copy(tid.at[pl.ds(base, BPW)], tid_v)
    pltpu.async_copy(ent.at[hid_v], head_v, semA0).wait()
    pltpu.async_copy(rel.at[rid_v], relr_v, semA0).wait()
    pltpu.async_copy(ent.at[tid_v], tail_v, semA0).wait()

    def pbody(b, carry):
        facc = jnp.zeros((L,), f32)
        for k in range(D // L):
            sl = pl.ds(k * L, L)
            h = head_v[b, sl]
            r = relr_v[b, sl]
            t = tail_v[b, sl]
            rt = r * t
            rt_v[b, sl] = rt
            hr_v[b, sl] = h * r
            facc = facc + h * rt
        factp_v[pl.ds(b * L, L)] = facc
        return carry

    lax.fori_loop(0, BPW, pbody, 0)
    gbase = [(iota + g * L) * L for g in range(BPW // L)]
    rot = [jnp.bitwise_and(iota + d, L - 1) for d in range(L)]
    for q in range(BPW // L):
        acc = jnp.zeros((L,), f32)
        for d in range(L):
            acc = acc + plsc.load_gather(factp_v, [gbase[q] + rot[d]])
        fact_v[pl.ds(q * L, L)] = acc
    pltpu.sync_copy(fact_v, fact_o.at[pl.ds(base, BPW)])

    bufs = ((rowsA0, semA0, rowsB0, semB0), (rowsA1, semA1, rowsB1, semB1))

    def fire(b, s):
        rA, sA, rB, sB = bufs[s]
        pltpu.async_copy(ent.at[nidxA.at[b]], rA, sA)
        pltpu.async_copy(ent.at[nidxB.at[b]], rB, sB)

    def drain(s):
        rA, sA, rB, sB = bufs[s]
        pltpu.make_async_copy(ent.at[nidxA.at[0]], rA, sA).wait()
        pltpu.make_async_copy(ent.at[nidxB.at[0]], rB, sB).wait()

    def side(neg, coef_v, out_hbm):
        pltpu.sync_copy(neg.at[pl.ds(base, BPW), pl.ds(0, CHUNK)], nidxA)
        pltpu.sync_copy(neg.at[pl.ds(base, BPW), pl.ds(CHUNK, CHUNK)], nidxB)
        fire(0, 0)

        def compute(b, s):
            rA, _, rB, _ = bufs[s]
            cvecs = [coef_v[b, pl.ds(k * L, L)] for k in range(D // L)]
            for j, rows in ((0, rA), (1, rB)):
                def rbody(n, carry, rows=rows):
                    facc = rows[n, pl.ds(0, L)] * cvecs[0]
                    for k in range(1, D // L):
                        facc = facc + rows[n, pl.ds(k * L, L)] * cvecs[k]
                    factp_v[pl.ds(n * L, L)] = facc
                    return carry

                lax.fori_loop(0, CHUNK, rbody, 0, unroll=8)
                accs = [jnp.zeros((L,), f32) for _ in range(CHUNK // L)]
                for d in range(L):
                    r = rot[d]
                    for g in range(CHUNK // L):
                        accs[g] = accs[g] + plsc.load_gather(
                            factp_v, [gbase[g] + r])
                for g in range(CHUNK // L):
                    score_v[b, pl.ds(j * CHUNK + g * L, L)] = accs[g]

        def pair(i, carry):
            b0 = 2 * i
            b1 = b0 + 1
            fire(b1, 1)
            drain(0)
            compute(b0, 0)
            fire(jnp.minimum(b1 + 1, BPW - 1), 0)
            drain(1)
            compute(b1, 1)
            return carry

        lax.fori_loop(0, BPW // 2, pair, 0)
        drain(0)
        pltpu.sync_copy(score_v, out_hbm.at[pl.ds(base, BPW), :])

    side(negh, rt_v, hs_o)
    side(negt, hr_v, ts_o)


def kernel(positive_samples, negative_heads, negative_tails,
           entity_embeddings, relation_embeddings):
    i32 = jnp.int32
    hid = positive_samples[:, 0].astype(i32)
    rid = positive_samples[:, 1].astype(i32)
    tid = positive_samples[:, 2].astype(i32)
    negh = negative_heads.astype(i32)
    negt = negative_tails.astype(i32)

    mesh = plsc.VectorSubcoreMesh(core_axis_name="c", subcore_axis_name="s")
    run = functools.partial(
        pl.kernel,
        mesh=mesh,
        compiler_params=pltpu.CompilerParams(
            needs_layout_passes=False, use_tc_tiling_on_sc=False),
        out_type=[
            jax.ShapeDtypeStruct((B,), jnp.float32),
            jax.ShapeDtypeStruct((B, NNEG), jnp.float32),
            jax.ShapeDtypeStruct((B, NNEG), jnp.float32),
        ],
        scratch_types=[
            pltpu.VMEM((BPW,), i32),
            pltpu.VMEM((BPW,), i32),
            pltpu.VMEM((BPW,), i32),
            pltpu.VMEM((BPW, D), jnp.float32),
            pltpu.VMEM((BPW, D), jnp.float32),
            pltpu.VMEM((BPW * L,), jnp.float32),
            pltpu.VMEM((BPW,), jnp.float32),
            pltpu.VMEM((BPW, CHUNK), i32),
            pltpu.VMEM((BPW, CHUNK), i32),
            pltpu.VMEM((CHUNK, D), jnp.float32),
            pltpu.VMEM((CHUNK, D), jnp.float32),
            pltpu.VMEM((CHUNK, D), jnp.float32),
            pltpu.VMEM((CHUNK, D), jnp.float32),
            pltpu.VMEM((BPW, NNEG), jnp.float32),
            pltpu.SemaphoreType.DMA,
            pltpu.SemaphoreType.DMA,
            pltpu.SemaphoreType.DMA,
            pltpu.SemaphoreType.DMA,
        ],
    )(_dist_mult_sc)

    fact, hs, ts = run(hid, rid, tid, negh, negt,
                       entity_embeddings, relation_embeddings)
    return (fact.reshape(B, 1), hs, ts)


# speedup vs baseline: 1.4621x; 1.2754x over previous
import functools

import jax
import jax.numpy as jnp
from jax import lax
from jax.experimental import layout as jex_layout
from jax.experimental import pallas as pl
from jax.experimental.pallas import tpu as pltpu
from jax.experimental.pallas import tpu_sc as plsc

B = 4096
NNEG = 256
D = 64
NCHUNK = 2
CHUNK = NNEG // NCHUNK
L = 16
NW = 32
BPW = B // NW


def _dist_mult_sc(hid, rid, tid, negh, negt, ent, rel,
                  fact_o, hs_o, ts_o,
                  hid_v, rid_v, tid_v,
                  rt_v, hr_v, factp_v, fact_v,
                  nidxA, nidxB, rowsA0, rowsB0, rowsA1, rowsB1, score_v,
                  semA0, semB0, semA1, semB1):
    wid = lax.axis_index("c") * 16 + lax.axis_index("s")
    base = wid * BPW

    f32 = jnp.float32
    i32 = jnp.int32
    iota = lax.iota(i32, L)

    head_v, relr_v, tail_v = rowsA0, rowsB0, rowsA1
    pltpu.sync_copy(hid.at[pl.ds(base, BPW)], hid_v)
    pltpu.sync_copy(rid.at[pl.ds(base, BPW)], rid_v)
    pltpu.sync_copy(tid.at[pl.ds(base, BPW)], tid_v)
    pltpu.async_copy(ent.at[hid_v], head_v, semA0).wait()
    pltpu.async_copy(rel.at[rid_v], relr_v, semA0).wait()
    pltpu.async_copy(ent.at[tid_v], tail_v, semA0).wait()

    def pbody(b, carry):
        facc = jnp.zeros((L,), f32)
        for k in range(D // L):
            sl = pl.ds(k * L, L)
            h = head_v[b, sl]
            r = relr_v[b, sl]
            t = tail_v[b, sl]
            rt = r * t
            rt_v[b, sl] = rt
            hr_v[b, sl] = h * r
            facc = facc + h * rt
        factp_v[pl.ds(b * L, L)] = facc
        return carry

    lax.fori_loop(0, BPW, pbody, 0)
    gbase = [(iota + g * L) * L for g in range(BPW // L)]
    rot = [jnp.bitwise_and(iota + d, L - 1) for d in range(L)]
    for q in range(BPW // L):
        acc = jnp.zeros((L,), f32)
        for d in range(L):
            acc = acc + plsc.load_gather(factp_v, [gbase[q] + rot[d]])
        fact_v[pl.ds(q * L, L)] = acc
    pltpu.sync_copy(fact_v, fact_o.at[pl.ds(base, BPW)])

    bufs = ((rowsA0, semA0, rowsB0, semB0), (rowsA1, semA1, rowsB1, semB1))

    def fire(b, s):
        rA, sA, rB, sB = bufs[s]
        pltpu.async_copy(ent.at[nidxA.at[b]], rA, sA)
        pltpu.async_copy(ent.at[nidxB.at[b]], rB, sB)

    def drain(s):
        rA, sA, rB, sB = bufs[s]
        pltpu.make_async_copy(ent.at[nidxA.at[0]], rA, sA).wait()
        pltpu.make_async_copy(ent.at[nidxB.at[0]], rB, sB).wait()

    def side(neg, coef_v, out_hbm):
        pltpu.sync_copy(neg.at[pl.ds(base, BPW), pl.ds(0, CHUNK)], nidxA)
        pltpu.sync_copy(neg.at[pl.ds(base, BPW), pl.ds(CHUNK, CHUNK)], nidxB)
        fire(0, 0)

        def compute(b, s):
            rA, _, rB, _ = bufs[s]
            cvecs = [coef_v[b, pl.ds(k * L, L)] for k in range(D // L)]
            for j, rows in ((0, rA), (1, rB)):
                def rbody(n, carry, rows=rows):
                    facc = rows[n, pl.ds(0, L)] * cvecs[0]
                    for k in range(1, D // L):
                        facc = facc + rows[n, pl.ds(k * L, L)] * cvecs[k]
                    factp_v[pl.ds(n * L, L)] = facc
                    return carry

                lax.fori_loop(0, CHUNK, rbody, 0, unroll=4)
                accs = [jnp.zeros((L,), f32) for _ in range(CHUNK // L)]
                for d in range(L):
                    r = rot[d]
                    for g in range(CHUNK // L):
                        accs[g] = accs[g] + plsc.load_gather(
                            factp_v, [gbase[g] + r])
                for g in range(CHUNK // L):
                    score_v[b, pl.ds(j * CHUNK + g * L, L)] = accs[g]

        def pair(i, carry):
            b0 = 2 * i
            b1 = b0 + 1
            fire(b1, 1)
            drain(0)
            compute(b0, 0)
            fire(jnp.minimum(b1 + 1, BPW - 1), 0)
            drain(1)
            compute(b1, 1)
            return carry

        lax.fori_loop(0, BPW // 2, pair, 0)
        drain(0)
        pltpu.sync_copy(score_v, out_hbm.at[pl.ds(base, BPW), :])

    side(negh, rt_v, hs_o)
    side(negt, hr_v, ts_o)


def kernel(positive_samples, negative_heads, negative_tails,
           entity_embeddings, relation_embeddings):
    i32 = jnp.int32
    entity_embeddings = jex_layout.with_layout_constraint(
        entity_embeddings,
        jex_layout.Layout(major_to_minor=(0, 1), tiling=((8, 128),)),
    )
    hid = positive_samples[:, 0].astype(i32)
    rid = positive_samples[:, 1].astype(i32)
    tid = positive_samples[:, 2].astype(i32)
    negh = negative_heads.astype(i32)
    negt = negative_tails.astype(i32)

    mesh = plsc.VectorSubcoreMesh(core_axis_name="c", subcore_axis_name="s")
    run = functools.partial(
        pl.kernel,
        mesh=mesh,
        compiler_params=pltpu.CompilerParams(
            needs_layout_passes=False, use_tc_tiling_on_sc=False),
        out_type=[
            jax.ShapeDtypeStruct((B,), jnp.float32),
            jax.ShapeDtypeStruct((B, NNEG), jnp.float32),
            jax.ShapeDtypeStruct((B, NNEG), jnp.float32),
        ],
        scratch_types=[
            pltpu.VMEM((BPW,), i32),
            pltpu.VMEM((BPW,), i32),
            pltpu.VMEM((BPW,), i32),
            pltpu.VMEM((BPW, D), jnp.float32),
            pltpu.VMEM((BPW, D), jnp.float32),
            pltpu.VMEM((BPW * L,), jnp.float32),
            pltpu.VMEM((BPW,), jnp.float32),
            pltpu.VMEM((BPW, CHUNK), i32),
            pltpu.VMEM((BPW, CHUNK), i32),
            pltpu.VMEM((CHUNK, D), jnp.float32),
            pltpu.VMEM((CHUNK, D), jnp.float32),
            pltpu.VMEM((CHUNK, D), jnp.float32),
            pltpu.VMEM((CHUNK, D), jnp.float32),
            pltpu.VMEM((BPW, NNEG), jnp.float32),
            pltpu.SemaphoreType.DMA,
            pltpu.SemaphoreType.DMA,
            pltpu.SemaphoreType.DMA,
            pltpu.SemaphoreType.DMA,
        ],
    )(_dist_mult_sc)

    fact, hs, ts = run(hid, rid, tid, negh, negt,
                       entity_embeddings, relation_embeddings)
    return (fact.reshape(B, 1), hs, ts)
